# Initial kernel scaffold; baseline (speedup 1.0000x reference)
import jax
import jax.numpy as jnp
from jax.experimental import pallas as pl


def kernel(features):
    raise NotImplementedError("write your pallas kernel here")


# speedup vs baseline: 11.6370x; 11.6370x over previous
import functools

import jax
import jax.numpy as jnp
from jax import lax
from jax.experimental import pallas as pl
from jax.experimental.pallas import tpu as pltpu

K = 64
THRESHOLD = 0.5
HI_BITS = 0x3F800000


def _deg_kernel(x_ref, deg_ref, *, b2):
    b = pl.program_id(0)
    x = x_ref[0]
    masked = jnp.where(x >= THRESHOLD, x, 0.0)

    @pl.when(b < b2)
    def _nor():
        deg_ref[0, 0, :] = jnp.sum(masked, axis=1)

    @pl.when(b >= b2)
    def _abn():
        xi = lax.bitcast_convert_type(masked, jnp.int32)
        c = x.shape[0]
        lo = jnp.zeros((c, 1), jnp.int32)
        hi = jnp.full((c, 1), HI_BITS, jnp.int32)

        def body(_, carry):
            lo, hi = carry
            mid = lo + ((hi - lo + 1) >> 1)
            cnt = jnp.sum((xi >= mid).astype(jnp.int32), axis=1, keepdims=True)
            ge = cnt >= K
            return jnp.where(ge, mid, lo), jnp.where(ge, hi, mid - 1)

        lo, hi = lax.fori_loop(0, 30, body, (lo, hi))
        kth_f = lax.bitcast_convert_type(lo, jnp.float32)
        gt = xi > lo
        s = jnp.sum(jnp.where(gt, masked, 0.0), axis=1, keepdims=True)
        cnt_gt = jnp.sum(gt.astype(jnp.float32), axis=1, keepdims=True)
        deg = s + (K - cnt_gt) * kth_f
        deg_ref[0, 0, :] = deg[:, 0]


def _loss_kernel(deg_ref, out_ref, *, b2):
    deg = deg_ref[...]
    b, c = deg.shape
    mean = jnp.mean(deg, axis=1, keepdims=True)
    d = deg - mean
    var = jnp.sum(d * d, axis=1, keepdims=True) / (c - 1)
    sign = jnp.where(lax.broadcasted_iota(jnp.int32, (b, 1), 0) < b2, 1.0, -1.0)
    out_ref[...] = jnp.sum(var * sign, axis=(0, 1), keepdims=True) / b2


def kernel(features):
    b, c, t = features.shape
    b2 = b // 2

    deg = pl.pallas_call(
        functools.partial(_deg_kernel, b2=b2),
        grid=(b,),
        in_specs=[pl.BlockSpec((1, c, t), lambda i: (i, 0, 0))],
        out_specs=pl.BlockSpec((1, 1, c), lambda i: (i, 0, 0)),
        out_shape=jax.ShapeDtypeStruct((b, 1, c), jnp.float32),
    )(features)
    deg = jnp.reshape(deg, (b, c))

    loss = pl.pallas_call(
        functools.partial(_loss_kernel, b2=b2),
        out_shape=jax.ShapeDtypeStruct((1, 1), jnp.float32),
    )(deg)
    return jnp.reshape(loss, ())
